# Initial kernel scaffold; baseline (speedup 1.0000x reference)
import jax
import jax.numpy as jnp
from jax.experimental import pallas as pl


def kernel(returns, positions, velocities, W1, b1, W2, b2, gamma, beta):
    raise NotImplementedError("write your pallas kernel here")


# speedup vs baseline: 8.7995x; 8.7995x over previous
import functools

import numpy as np
import jax
import jax.numpy as jnp
from jax.experimental import pallas as pl

_N = 100
_RADIUS = 0.1
_DT = 0.01
_PAD = 128

_SIGMA = np.array([8 * (u % 16) + u // 16 for u in range(_PAD)], dtype=np.int64)
_PSIG_NP = np.zeros((_PAD, _PAD), dtype=np.float32)
_PSIG_NP[np.arange(_PAD), _SIGMA] = 1.0


def _bf16_split3(x):
    x1 = x.astype(jnp.bfloat16).astype(jnp.float32)
    r1 = x - x1
    x2 = r1.astype(jnp.bfloat16).astype(jnp.float32)
    x3 = r1 - x2
    return x1, x2, x3


def _tree_reduce(R):
    A0 = R[0:16] + R[16:32]
    A1 = R[32:48] + R[48:64]
    A2 = R[64:80] + R[80:96]
    A3 = R[96:112] + R[112:128]
    B0 = A0 + A1
    B1 = A2 + A3
    P = B0 + B1
    acc = P[0:1]
    for t in range(1, 16):
        acc = acc + P[t:t + 1]
    return acc


def _sim_kernel(nsteps, psig_ref, pos_ref, vel_ref, w1t_ref, b1_ref, w2t_ref,
                b2_ref, gam_ref, bet_ref, feats_ref, enc_ref):
    psig = psig_ref[...]
    ident = (jax.lax.broadcasted_iota(jnp.int32, (_PAD, _PAD), 0) ==
             jax.lax.broadcasted_iota(jnp.int32, (_PAD, _PAD), 1))
    identf = ident.astype(jnp.float32)
    jlane = jax.lax.broadcasted_iota(jnp.int32, (1, _PAD), 1)
    isub = jax.lax.broadcasted_iota(jnp.int32, (_PAD, 1), 0)
    validr = jlane < _N
    validc = isub < _N
    vpair = validc & validr
    oh0 = (jlane == 0).astype(jnp.float32)
    oh1 = (jlane == 1).astype(jnp.float32)
    oh2 = (jlane == 2).astype(jnp.float32)

    def step(b, carry):
        posC, velC = carry
        px = posC[:, 0:1]
        py = posC[:, 1:2]
        pxr = jnp.sum(px * identf, axis=0, keepdims=True)
        pyr = jnp.sum(py * identf, axis=0, keepdims=True)
        dx = px - pxr
        dy = py - pyr
        d2 = dx * dx + dy * dy
        nz = d2 > 0
        dist = jnp.where(nz, jnp.sqrt(jnp.where(nz, d2, 1.0)), 0.0)
        mask = (dist < _RADIUS) & (dist > 0)
        mf = mask.astype(jnp.float32)
        cnt = jnp.sum(mf, axis=1, keepdims=True)
        safe = jnp.maximum(cnt, 1.0)
        avgv = jnp.dot(mf, velC) / safe
        avgp = jnp.dot(mf, posC) / safe
        has = cnt > 0
        align = jnp.where(has, avgv - velC, 0.0)
        coh = jnp.where(has, avgp - posC, 0.0)

        x1, x2, x3 = _bf16_split3(px)
        y1, y2, y3 = _bf16_split3(py)
        S = jnp.concatenate([x1, x2, x3, y1, y2, y3], axis=1)
        Sp = jnp.dot(psig, S)
        pxs = Sp[:, 0:1] + Sp[:, 1:2] + Sp[:, 2:3]
        pys = Sp[:, 3:4] + Sp[:, 4:5] + Sp[:, 5:6]
        dxs = pxs - pxr
        dys = pys - pyr
        d2s = dxs * dxs + dys * dys
        nzs = d2s > 0
        dists = jnp.where(nzs, jnp.sqrt(jnp.where(nzs, d2s, 1.0)), 0.0)
        masks = (dists < _RADIUS) & (dists > 0)
        invs = jnp.where(masks, 1.0 / jnp.where(nzs, d2s, 1.0), 0.0)
        repx = _tree_reduce(-(dxs * invs))
        repy = _tree_reduce(-(dys * invs))
        repx_c = jnp.sum(repx * identf, axis=1, keepdims=True)
        repy_c = jnp.sum(repy * identf, axis=1, keepdims=True)

        accx = align[:, 0:1] + coh[:, 0:1] + repx_c
        accy = align[:, 1:2] + coh[:, 1:2] + repy_c
        nvx = velC[:, 0:1] + accx * _DT
        nvy = velC[:, 1:2] + accy * _DT
        s2 = nvx * nvx + nvy * nvy
        snz = s2 > 0
        speed = jnp.where(snz, jnp.sqrt(jnp.where(snz, s2, 1.0)), 0.0)
        fac = 0.1 / jnp.maximum(speed, 1e-12)
        cl = speed > 0.1
        nvx = jnp.where(cl, nvx * fac, nvx)
        nvy = jnp.where(cl, nvy * fac, nvy)
        npx = px + nvx * _DT
        npy = py + nvy * _DT

        npxr = jnp.sum(npx * identf, axis=0, keepdims=True)
        npyr = jnp.sum(npy * identf, axis=0, keepdims=True)
        dxm = npx - npxr
        dym = npy - npyr
        d2m = dxm * dxm + dym * dym
        mnzm = d2m > 0
        dm = jnp.where(mnzm, jnp.sqrt(jnp.where(mnzm, d2m, 1.0)), 0.0)
        mvx = jnp.sum(nvx) / _N
        mvy = jnp.sum(nvy) / _N
        pol = jnp.sqrt(mvx * mvx + mvy * mvy) / 0.1
        dmsum = jnp.sum(jnp.where(vpair, dm, 0.0))
        clus = 1.0 / (1.0 + dmsum / (_N * _N))
        reach = (((dm < _RADIUS) & vpair) | ident).astype(jnp.float32)
        for _ in range(7):
            reach = (jnp.dot(reach, reach) > 0).astype(jnp.float32)
        minj = jnp.min(jnp.where(reach > 0, jlane, _PAD), axis=1, keepdims=True)
        ncomp = jnp.sum(((minj == isub) & validc).astype(jnp.float32))
        frag = ncomp / _N

        feats_ref[pl.ds(b, 1), :] = pol * oh0 + clus * oh1 + frag * oh2
        posN = jnp.concatenate([npx, npy], axis=1)
        velN = jnp.concatenate([nvx, nvy], axis=1)
        return (posN, velN)

    jax.lax.fori_loop(0, nsteps, step, (pos_ref[...], vel_ref[...]))

    feats = feats_ref[...]
    h = jnp.dot(feats, w1t_ref[...])
    h = jnp.maximum(h + b1_ref[...], 0.0)
    h = jnp.dot(h, w2t_ref[...]) + b2_ref[...]
    mu = jnp.mean(h, axis=-1, keepdims=True)
    var = jnp.mean((h - mu) ** 2, axis=-1, keepdims=True)
    enc_ref[...] = (h - mu) / jnp.sqrt(var + 1e-5) * gam_ref[...] + bet_ref[...]


def kernel(returns, positions, velocities, W1, b1, W2, b2, gamma, beta):
    B = returns.shape[0]
    posP = jnp.full((_PAD, 2), 1e9, jnp.float32).at[:_N].set(positions)
    velP = jnp.zeros((_PAD, 2), jnp.float32).at[:_N].set(velocities)
    psig = jnp.asarray(_PSIG_NP)
    w1t = jnp.zeros((_PAD, 64), jnp.float32).at[0:3].set(W1.T)
    feats, enc = pl.pallas_call(
        functools.partial(_sim_kernel, B),
        out_shape=(jax.ShapeDtypeStruct((B, _PAD), jnp.float32),
                   jax.ShapeDtypeStruct((B, 32), jnp.float32)),
    )(psig, posP, velP, w1t, b1.reshape(1, 64), W2.T, b2.reshape(1, 32),
      gamma.reshape(1, 32), beta.reshape(1, 32))
    return feats[:, 0], feats[:, 1], feats[:, 2], enc


# speedup vs baseline: 9.8720x; 1.1219x over previous
import functools

import numpy as np
import jax
import jax.numpy as jnp
from jax.experimental import pallas as pl
from jax.experimental.pallas import tpu as pltpu

_N = 100
_RADIUS = 0.1
_DT = 0.01
_PAD = 128
_UNROLL = 4

_SIGMA = np.array([8 * (u % 16) + u // 16 for u in range(_PAD)], dtype=np.int64)
_PSIG_NP = np.zeros((_PAD, _PAD), dtype=np.float32)
_PSIG_NP[np.arange(_PAD), _SIGMA] = 1.0


def _bf16_split3(x):
    x1 = x.astype(jnp.bfloat16).astype(jnp.float32)
    r1 = x - x1
    x2 = r1.astype(jnp.bfloat16).astype(jnp.float32)
    x3 = r1 - x2
    return x1, x2, x3


def _tree_reduce(R):
    A0 = R[0:16] + R[16:32]
    A1 = R[32:48] + R[48:64]
    A2 = R[64:80] + R[80:96]
    A3 = R[96:112] + R[112:128]
    B0 = A0 + A1
    B1 = A2 + A3
    P = B0 + B1
    acc = P[0:1]
    for t in range(1, 16):
        acc = acc + P[t:t + 1]
    return acc


def _sim_kernel(nsteps, psig_ref, pos_ref, vel_ref, w1t_ref, b1_ref, w2t_ref,
                b2_ref, gam_ref, bet_ref, feats_ref, enc_ref, px_hist, py_hist):
    psig = psig_ref[...]
    ident = (jax.lax.broadcasted_iota(jnp.int32, (_PAD, _PAD), 0) ==
             jax.lax.broadcasted_iota(jnp.int32, (_PAD, _PAD), 1))
    jlane = jax.lax.broadcasted_iota(jnp.int32, (1, _PAD), 1)
    isub = jax.lax.broadcasted_iota(jnp.int32, (_PAD, 1), 0)
    validr = jlane < _N
    validc = isub < _N
    vpair = validc & validr
    oh0 = (jlane == 0).astype(jnp.float32)
    oh1 = (jlane == 1).astype(jnp.float32)
    oh2 = (jlane == 2).astype(jnp.float32)

    def step(b, carry):
        posC, velC, pxr, pyr = carry
        px = posC[:, 0:1]
        py = posC[:, 1:2]
        dx = px - pxr
        dy = py - pyr
        d2 = dx * dx + dy * dy
        nz = d2 > 0
        dist = jnp.where(nz, jnp.sqrt(jnp.where(nz, d2, 1.0)), 0.0)
        mask = (dist < _RADIUS) & (dist > 0)
        mf = mask.astype(jnp.float32)
        cnt = jnp.sum(mf, axis=1, keepdims=True)
        safe = jnp.maximum(cnt, 1.0)
        vp4 = jnp.concatenate([velC, posC], axis=1)
        avg = jnp.dot(mf, vp4) / safe
        has = cnt > 0
        align = jnp.where(has, avg[:, 0:2] - velC, 0.0)
        coh = jnp.where(has, avg[:, 2:4] - posC, 0.0)

        x1, x2, x3 = _bf16_split3(px)
        y1, y2, y3 = _bf16_split3(py)
        S = jnp.concatenate([x1, x2, x3, y1, y2, y3], axis=1)
        Sp = jnp.dot(psig, S)
        pxs = Sp[:, 0:1] + Sp[:, 1:2] + Sp[:, 2:3]
        pys = Sp[:, 3:4] + Sp[:, 4:5] + Sp[:, 5:6]
        dxs = pxs - pxr
        dys = pys - pyr
        d2s = dxs * dxs + dys * dys
        nzs = d2s > 0
        dists = jnp.where(nzs, jnp.sqrt(jnp.where(nzs, d2s, 1.0)), 0.0)
        masks = (dists < _RADIUS) & (dists > 0)
        invs = jnp.where(masks, 1.0 / jnp.where(nzs, d2s, 1.0), 0.0)
        repx = _tree_reduce(-(dxs * invs))
        repy = _tree_reduce(-(dys * invs))
        repx_c = jnp.swapaxes(repx, 0, 1)
        repy_c = jnp.swapaxes(repy, 0, 1)

        accx = align[:, 0:1] + coh[:, 0:1] + repx_c
        accy = align[:, 1:2] + coh[:, 1:2] + repy_c
        nvx = velC[:, 0:1] + accx * _DT
        nvy = velC[:, 1:2] + accy * _DT
        s2 = nvx * nvx + nvy * nvy
        snz = s2 > 0
        speed = jnp.where(snz, jnp.sqrt(jnp.where(snz, s2, 1.0)), 0.0)
        fac = 0.1 / jnp.maximum(speed, 1e-12)
        cl = speed > 0.1
        nvx = jnp.where(cl, nvx * fac, nvx)
        nvy = jnp.where(cl, nvy * fac, nvy)
        npx = px + nvx * _DT
        npy = py + nvy * _DT
        npxr = jnp.swapaxes(npx, 0, 1)
        npyr = jnp.swapaxes(npy, 0, 1)
        px_hist[pl.ds(b, 1), :] = npxr
        py_hist[pl.ds(b, 1), :] = npyr

        mvx = jnp.sum(nvx) / _N
        mvy = jnp.sum(nvy) / _N
        pol = jnp.sqrt(mvx * mvx + mvy * mvy) / 0.1
        feats_ref[pl.ds(b, 1), :] = pol * oh0

        posN = jnp.concatenate([npx, npy], axis=1)
        velN = jnp.concatenate([nvx, nvy], axis=1)
        return (posN, velN, npxr, npyr)

    posC0 = pos_ref[...]
    velC0 = vel_ref[...]
    pxr0 = jnp.swapaxes(posC0[:, 0:1], 0, 1)
    pyr0 = jnp.swapaxes(posC0[:, 1:2], 0, 1)
    jax.lax.fori_loop(0, nsteps, step, (posC0, velC0, pxr0, pyr0))

    def metrics_one(b):
        pxr = px_hist[pl.ds(b, 1), :]
        pyr = py_hist[pl.ds(b, 1), :]
        px = jnp.swapaxes(pxr, 0, 1)
        py = jnp.swapaxes(pyr, 0, 1)
        dxm = px - pxr
        dym = py - pyr
        d2m = dxm * dxm + dym * dym
        mnzm = d2m > 0
        dm = jnp.where(mnzm, jnp.sqrt(jnp.where(mnzm, d2m, 1.0)), 0.0)
        dmsum = jnp.sum(jnp.where(vpair, dm, 0.0))
        clus = 1.0 / (1.0 + dmsum / (_N * _N))
        reach = (((dm < _RADIUS) & vpair) | ident).astype(jnp.float32)
        for _ in range(7):
            reach = (jnp.dot(reach, reach) > 0).astype(jnp.float32)
        minj = jnp.min(jnp.where(reach > 0, jlane, _PAD), axis=1, keepdims=True)
        ncomp = jnp.sum(((minj == isub) & validc).astype(jnp.float32))
        frag = ncomp / _N
        feats_ref[pl.ds(b, 1), :] = feats_ref[pl.ds(b, 1), :] + clus * oh1 + frag * oh2

    def metrics_group(g, _):
        for u in range(_UNROLL):
            metrics_one(g * _UNROLL + u)
        return 0

    jax.lax.fori_loop(0, nsteps // _UNROLL, metrics_group, 0)
    for b in range((nsteps // _UNROLL) * _UNROLL, nsteps):
        metrics_one(b)

    feats = feats_ref[...]
    h = jnp.dot(feats, w1t_ref[...])
    h = jnp.maximum(h + b1_ref[...], 0.0)
    h = jnp.dot(h, w2t_ref[...]) + b2_ref[...]
    mu = jnp.mean(h, axis=-1, keepdims=True)
    var = jnp.mean((h - mu) ** 2, axis=-1, keepdims=True)
    enc_ref[...] = (h - mu) / jnp.sqrt(var + 1e-5) * gam_ref[...] + bet_ref[...]


def kernel(returns, positions, velocities, W1, b1, W2, b2, gamma, beta):
    B = returns.shape[0]
    posP = jnp.full((_PAD, 2), 1e9, jnp.float32).at[:_N].set(positions)
    velP = jnp.zeros((_PAD, 2), jnp.float32).at[:_N].set(velocities)
    psig = jnp.asarray(_PSIG_NP)
    w1t = jnp.zeros((_PAD, 64), jnp.float32).at[0:3].set(W1.T)
    feats, enc = pl.pallas_call(
        functools.partial(_sim_kernel, B),
        out_shape=(jax.ShapeDtypeStruct((B, _PAD), jnp.float32),
                   jax.ShapeDtypeStruct((B, 32), jnp.float32)),
        scratch_shapes=[pltpu.VMEM((B, _PAD), jnp.float32),
                        pltpu.VMEM((B, _PAD), jnp.float32)],
    )(psig, posP, velP, w1t, b1.reshape(1, 64), W2.T, b2.reshape(1, 32),
      gamma.reshape(1, 32), beta.reshape(1, 32))
    return feats[:, 0], feats[:, 1], feats[:, 2], enc


# speedup vs baseline: 9.9745x; 1.0104x over previous
import functools

import numpy as np
import jax
import jax.numpy as jnp
from jax.experimental import pallas as pl
from jax.experimental.pallas import tpu as pltpu

_N = 100
_RADIUS = 0.1
_DT = 0.01
_PAD = 128
_UNROLL = 8

_SIGMA = np.array([8 * (u % 16) + u // 16 for u in range(_PAD)], dtype=np.int64)
_PSIG_NP = np.zeros((_PAD, _PAD), dtype=np.float32)
_PSIG_NP[np.arange(_PAD), _SIGMA] = 1.0


def _bf16_split3(x):
    x1 = x.astype(jnp.bfloat16).astype(jnp.float32)
    r1 = x - x1
    x2 = r1.astype(jnp.bfloat16).astype(jnp.float32)
    x3 = r1 - x2
    return x1, x2, x3


def _tree_reduce(R):
    A0 = R[0:16] + R[16:32]
    A1 = R[32:48] + R[48:64]
    A2 = R[64:80] + R[80:96]
    A3 = R[96:112] + R[112:128]
    B0 = A0 + A1
    B1 = A2 + A3
    P = B0 + B1
    acc = P[0:1]
    for t in range(1, 16):
        acc = acc + P[t:t + 1]
    return acc


def _sim_kernel(nsteps, psig_ref, pos_ref, vel_ref, w1t_ref, b1_ref, w2t_ref,
                b2_ref, gam_ref, bet_ref, feats_ref, enc_ref, px_hist, py_hist):
    psig = psig_ref[...]
    ident = (jax.lax.broadcasted_iota(jnp.int32, (_PAD, _PAD), 0) ==
             jax.lax.broadcasted_iota(jnp.int32, (_PAD, _PAD), 1))
    jlane = jax.lax.broadcasted_iota(jnp.int32, (1, _PAD), 1)
    isub = jax.lax.broadcasted_iota(jnp.int32, (_PAD, 1), 0)
    validr = jlane < _N
    validc = isub < _N
    vpair = validc & validr
    oh0 = (jlane == 0).astype(jnp.float32)
    oh1 = (jlane == 1).astype(jnp.float32)
    oh2 = (jlane == 2).astype(jnp.float32)

    def step(b, carry):
        posC, velC, pxr, pyr = carry
        px = posC[:, 0:1]
        py = posC[:, 1:2]
        dx = px - pxr
        dy = py - pyr
        d2 = dx * dx + dy * dy
        nz = d2 > 0
        dist = jnp.where(nz, jnp.sqrt(jnp.where(nz, d2, 1.0)), 0.0)
        mask = (dist < _RADIUS) & (dist > 0)
        mf = mask.astype(jnp.float32)
        cnt = jnp.sum(mf, axis=1, keepdims=True)
        safe = jnp.maximum(cnt, 1.0)
        vp4 = jnp.concatenate([velC, posC], axis=1)
        avg = jnp.dot(mf, vp4) / safe
        has = cnt > 0
        align = jnp.where(has, avg[:, 0:2] - velC, 0.0)
        coh = jnp.where(has, avg[:, 2:4] - posC, 0.0)

        x1, x2, x3 = _bf16_split3(px)
        y1, y2, y3 = _bf16_split3(py)
        S = jnp.concatenate([x1, x2, x3, y1, y2, y3], axis=1)
        Sp = jnp.dot(psig, S)
        pxs = Sp[:, 0:1] + Sp[:, 1:2] + Sp[:, 2:3]
        pys = Sp[:, 3:4] + Sp[:, 4:5] + Sp[:, 5:6]
        dxs = pxs - pxr
        dys = pys - pyr
        d2s = dxs * dxs + dys * dys
        nzs = d2s > 0
        dists = jnp.where(nzs, jnp.sqrt(jnp.where(nzs, d2s, 1.0)), 0.0)
        masks = (dists < _RADIUS) & (dists > 0)
        invs = jnp.where(masks, 1.0 / jnp.where(nzs, d2s, 1.0), 0.0)
        repx = _tree_reduce(-(dxs * invs))
        repy = _tree_reduce(-(dys * invs))
        repx_c = jnp.swapaxes(repx, 0, 1)
        repy_c = jnp.swapaxes(repy, 0, 1)

        accx = align[:, 0:1] + coh[:, 0:1] + repx_c
        accy = align[:, 1:2] + coh[:, 1:2] + repy_c
        nvx = velC[:, 0:1] + accx * _DT
        nvy = velC[:, 1:2] + accy * _DT
        s2 = nvx * nvx + nvy * nvy
        snz = s2 > 0
        speed = jnp.where(snz, jnp.sqrt(jnp.where(snz, s2, 1.0)), 0.0)
        fac = 0.1 / jnp.maximum(speed, 1e-12)
        cl = speed > 0.1
        nvx = jnp.where(cl, nvx * fac, nvx)
        nvy = jnp.where(cl, nvy * fac, nvy)
        npx = px + nvx * _DT
        npy = py + nvy * _DT
        npxr = jnp.swapaxes(npx, 0, 1)
        npyr = jnp.swapaxes(npy, 0, 1)
        px_hist[pl.ds(b + 1, 1), :] = npxr
        py_hist[pl.ds(b + 1, 1), :] = npyr

        posN = jnp.concatenate([npx, npy], axis=1)
        velN = jnp.concatenate([nvx, nvy], axis=1)
        return (posN, velN, npxr, npyr)

    posC0 = pos_ref[...]
    velC0 = vel_ref[...]
    pxr0 = jnp.swapaxes(posC0[:, 0:1], 0, 1)
    pyr0 = jnp.swapaxes(posC0[:, 1:2], 0, 1)
    px_hist[pl.ds(0, 1), :] = pxr0
    py_hist[pl.ds(0, 1), :] = pyr0
    jax.lax.fori_loop(0, nsteps, step, (posC0, velC0, pxr0, pyr0))

    def metrics_one(b):
        pxr0m = px_hist[pl.ds(b, 1), :]
        pyr0m = py_hist[pl.ds(b, 1), :]
        pxr = px_hist[pl.ds(b + 1, 1), :]
        pyr = py_hist[pl.ds(b + 1, 1), :]
        nvxr = (pxr - pxr0m) * (1.0 / _DT)
        nvyr = (pyr - pyr0m) * (1.0 / _DT)
        mvx = jnp.sum(jnp.where(validr, nvxr, 0.0)) / _N
        mvy = jnp.sum(jnp.where(validr, nvyr, 0.0)) / _N
        pol = jnp.sqrt(mvx * mvx + mvy * mvy) / 0.1

        px = jnp.swapaxes(pxr, 0, 1)
        py = jnp.swapaxes(pyr, 0, 1)
        dxm = px - pxr
        dym = py - pyr
        d2m = dxm * dxm + dym * dym
        mnzm = d2m > 0
        dm = jnp.where(mnzm, jnp.sqrt(jnp.where(mnzm, d2m, 1.0)), 0.0)
        dmsum = jnp.sum(jnp.where(vpair, dm, 0.0))
        clus = 1.0 / (1.0 + dmsum / (_N * _N))

        adj = (((dm < _RADIUS) & vpair) | ident).astype(jnp.float32)
        s1 = jnp.dot(adj, adj)
        s2m = jnp.dot(s1, s1)
        t1 = (s2m > 0).astype(jnp.float32)
        s3 = jnp.dot(t1, t1)
        s4 = jnp.dot(s3, s3)
        t2 = (s4 > 0).astype(jnp.float32)
        s5 = jnp.dot(t2, t2)
        s6 = jnp.dot(s5, s5)
        t3 = (s6 > 0).astype(jnp.float32)
        s7 = jnp.dot(t3, t3)
        minj = jnp.min(jnp.where(s7 > 0, jlane, _PAD), axis=1, keepdims=True)
        ncomp = jnp.sum(((minj == isub) & validc).astype(jnp.float32))
        frag = ncomp / _N
        feats_ref[pl.ds(b, 1), :] = pol * oh0 + clus * oh1 + frag * oh2

    def metrics_group(g, _):
        for u in range(_UNROLL):
            metrics_one(g * _UNROLL + u)
        return 0

    jax.lax.fori_loop(0, nsteps // _UNROLL, metrics_group, 0)
    for b in range((nsteps // _UNROLL) * _UNROLL, nsteps):
        metrics_one(b)

    feats = feats_ref[...]
    h = jnp.dot(feats, w1t_ref[...])
    h = jnp.maximum(h + b1_ref[...], 0.0)
    h = jnp.dot(h, w2t_ref[...]) + b2_ref[...]
    mu = jnp.mean(h, axis=-1, keepdims=True)
    var = jnp.mean((h - mu) ** 2, axis=-1, keepdims=True)
    enc_ref[...] = (h - mu) / jnp.sqrt(var + 1e-5) * gam_ref[...] + bet_ref[...]


def kernel(returns, positions, velocities, W1, b1, W2, b2, gamma, beta):
    B = returns.shape[0]
    posP = jnp.full((_PAD, 2), 1e9, jnp.float32).at[:_N].set(positions)
    velP = jnp.zeros((_PAD, 2), jnp.float32).at[:_N].set(velocities)
    psig = jnp.asarray(_PSIG_NP)
    w1t = jnp.zeros((_PAD, 64), jnp.float32).at[0:3].set(W1.T)
    feats, enc = pl.pallas_call(
        functools.partial(_sim_kernel, B),
        out_shape=(jax.ShapeDtypeStruct((B, _PAD), jnp.float32),
                   jax.ShapeDtypeStruct((B, 32), jnp.float32)),
        scratch_shapes=[pltpu.VMEM((B + 8, _PAD), jnp.float32),
                        pltpu.VMEM((B + 8, _PAD), jnp.float32)],
    )(psig, posP, velP, w1t, b1.reshape(1, 64), W2.T, b2.reshape(1, 32),
      gamma.reshape(1, 32), beta.reshape(1, 32))
    return feats[:, 0], feats[:, 1], feats[:, 2], enc


# speedup vs baseline: 12.4413x; 1.2473x over previous
import functools

import numpy as np
import jax
import jax.numpy as jnp
from jax.experimental import pallas as pl
from jax.experimental.pallas import tpu as pltpu

_N = 100
_RADIUS = 0.1
_DT = 0.01
_PAD = 128
_UNROLL = 8

_SIGMA = np.array([8 * (u % 16) + u // 16 for u in range(_PAD)], dtype=np.int64)
_PSIG_NP = np.zeros((_PAD, _PAD), dtype=np.float32)
_PSIG_NP[np.arange(_PAD), _SIGMA] = 1.0


def _bf16_split3(x):
    x1 = x.astype(jnp.bfloat16).astype(jnp.float32)
    r1 = x - x1
    x2 = r1.astype(jnp.bfloat16).astype(jnp.float32)
    x3 = r1 - x2
    return x1, x2, x3


def _tree_reduce(R):
    A0 = R[0:16] + R[16:32]
    A1 = R[32:48] + R[48:64]
    A2 = R[64:80] + R[80:96]
    A3 = R[96:112] + R[112:128]
    B0 = A0 + A1
    B1 = A2 + A3
    P = B0 + B1
    acc = P[0:1]
    for t in range(1, 16):
        acc = acc + P[t:t + 1]
    return acc


def _sim_kernel(nsteps, psig_ref, pos_ref, vel_ref, w1t_ref, b1_ref, w2t_ref,
                b2_ref, gam_ref, bet_ref, feats_ref, enc_ref, px_hist, py_hist):
    psig = psig_ref[...]
    ident = (jax.lax.broadcasted_iota(jnp.int32, (_PAD, _PAD), 0) ==
             jax.lax.broadcasted_iota(jnp.int32, (_PAD, _PAD), 1))
    jlane = jax.lax.broadcasted_iota(jnp.int32, (1, _PAD), 1)
    isub = jax.lax.broadcasted_iota(jnp.int32, (_PAD, 1), 0)
    validr = jlane < _N
    validc = isub < _N
    vpair = validc & validr
    oh0 = (jlane == 0).astype(jnp.float32)
    oh1 = (jlane == 1).astype(jnp.float32)
    oh2 = (jlane == 2).astype(jnp.float32)

    def metrics_one(b):
        pxr0m = px_hist[pl.ds(b, 1), :]
        pyr0m = py_hist[pl.ds(b, 1), :]
        pxr = px_hist[pl.ds(b + 1, 1), :]
        pyr = py_hist[pl.ds(b + 1, 1), :]
        nvxr = (pxr - pxr0m) * (1.0 / _DT)
        nvyr = (pyr - pyr0m) * (1.0 / _DT)
        mvx = jnp.sum(jnp.where(validr, nvxr, 0.0)) / _N
        mvy = jnp.sum(jnp.where(validr, nvyr, 0.0)) / _N
        pol = jnp.sqrt(mvx * mvx + mvy * mvy) / 0.1

        px = jnp.swapaxes(pxr, 0, 1)
        py = jnp.swapaxes(pyr, 0, 1)
        dxm = px - pxr
        dym = py - pyr
        d2m = dxm * dxm + dym * dym
        mnzm = d2m > 0
        dist = jnp.where(mnzm, jnp.sqrt(jnp.where(mnzm, d2m, 1.0)), 0.0)
        lt = dist < _RADIUS
        dmsum = jnp.sum(jnp.where(vpair, dist, 0.0))
        clus = 1.0 / (1.0 + dmsum / (_N * _N))
        adj = ((lt & vpair) | ident).astype(jnp.float32)
        s1 = jnp.dot(adj, adj)
        s2m = jnp.dot(s1, s1)
        t1 = (s2m > 0).astype(jnp.float32)
        s3 = jnp.dot(t1, t1)
        s4 = jnp.dot(s3, s3)
        t2 = (s4 > 0).astype(jnp.float32)
        s5 = jnp.dot(t2, t2)
        s6 = jnp.dot(s5, s5)
        t3 = (s6 > 0).astype(jnp.float32)
        s7 = jnp.dot(t3, t3)
        minj = jnp.min(jnp.where(s7 > 0, jlane, _PAD), axis=1, keepdims=True)
        ncomp = jnp.sum(((minj == isub) & validc).astype(jnp.float32))
        frag = ncomp / _N
        feats_ref[pl.ds(b, 1), :] = pol * oh0 + clus * oh1 + frag * oh2

    def step(b, carry):
        px, py, vx, vy, pxr, pyr = carry
        dx = px - pxr
        dy = py - pyr
        d2 = dx * dx + dy * dy
        nz = d2 > 0
        dist = jnp.where(nz, jnp.sqrt(jnp.where(nz, d2, 1.0)), 0.0)
        mask = (dist < _RADIUS) & (dist > 0)
        mf = mask.astype(jnp.float32)
        cnt = jnp.sum(mf, axis=1, keepdims=True)
        safe = jnp.maximum(cnt, 1.0)
        vp4 = jnp.concatenate([vx, vy, px, py], axis=1)
        avg = jnp.dot(mf, vp4) / safe
        has = cnt > 0
        alignx = jnp.where(has, avg[:, 0:1] - vx, 0.0)
        aligny = jnp.where(has, avg[:, 1:2] - vy, 0.0)
        cohx = jnp.where(has, avg[:, 2:3] - px, 0.0)
        cohy = jnp.where(has, avg[:, 3:4] - py, 0.0)

        x1, x2, x3 = _bf16_split3(px)
        y1, y2, y3 = _bf16_split3(py)
        S = jnp.concatenate([x1, x2, x3, y1, y2, y3], axis=1)
        Sp = jnp.dot(psig, S)
        pxs = Sp[:, 0:1] + Sp[:, 1:2] + Sp[:, 2:3]
        pys = Sp[:, 3:4] + Sp[:, 4:5] + Sp[:, 5:6]
        dxs = pxs - pxr
        dys = pys - pyr
        d2s = dxs * dxs + dys * dys
        nzs = d2s > 0
        dists = jnp.where(nzs, jnp.sqrt(jnp.where(nzs, d2s, 1.0)), 0.0)
        masks = (dists < _RADIUS) & (dists > 0)
        invs = jnp.where(masks, 1.0 / jnp.where(nzs, d2s, 1.0), 0.0)
        repx = _tree_reduce(-(dxs * invs))
        repy = _tree_reduce(-(dys * invs))
        rp = jnp.concatenate([repx, repy], axis=0)
        rpc = jnp.swapaxes(rp, 0, 1)

        accx = alignx + cohx + rpc[:, 0:1]
        accy = aligny + cohy + rpc[:, 1:2]
        nvx = vx + accx * _DT
        nvy = vy + accy * _DT
        s2 = nvx * nvx + nvy * nvy
        snz = s2 > 0
        speed = jnp.where(snz, jnp.sqrt(jnp.where(snz, s2, 1.0)), 0.0)
        fac = 0.1 / jnp.maximum(speed, 1e-12)
        cl = speed > 0.1
        nvx = jnp.where(cl, nvx * fac, nvx)
        nvy = jnp.where(cl, nvy * fac, nvy)
        npx = px + nvx * _DT
        npy = py + nvy * _DT
        np2 = jnp.concatenate([npx, npy], axis=1)
        np2r = jnp.swapaxes(np2, 0, 1)
        npxr = np2r[0:1]
        npyr = np2r[1:2]
        px_hist[pl.ds(b + 1, 1), :] = npxr
        py_hist[pl.ds(b + 1, 1), :] = npyr
        return (npx, npy, nvx, nvy, npxr, npyr)

    posC0 = pos_ref[...]
    velC0 = vel_ref[...]
    pxr0 = jnp.swapaxes(posC0[:, 0:1], 0, 1)
    pyr0 = jnp.swapaxes(posC0[:, 1:2], 0, 1)
    px_hist[pl.ds(0, 1), :] = pxr0
    py_hist[pl.ds(0, 1), :] = pyr0
    jax.lax.fori_loop(0, nsteps, step,
                      (posC0[:, 0:1], posC0[:, 1:2], velC0[:, 0:1],
                       velC0[:, 1:2], pxr0, pyr0))

    def metrics_group(g, _):
        for u in range(_UNROLL):
            metrics_one(g * _UNROLL + u)
        return 0

    jax.lax.fori_loop(0, nsteps // _UNROLL, metrics_group, 0)
    for b in range((nsteps // _UNROLL) * _UNROLL, nsteps):
        metrics_one(b)

    feats = feats_ref[...]
    h = jnp.dot(feats, w1t_ref[...])
    h = jnp.maximum(h + b1_ref[...], 0.0)
    h = jnp.dot(h, w2t_ref[...]) + b2_ref[...]
    mu = jnp.mean(h, axis=-1, keepdims=True)
    var = jnp.mean((h - mu) ** 2, axis=-1, keepdims=True)
    enc_ref[...] = (h - mu) / jnp.sqrt(var + 1e-5) * gam_ref[...] + bet_ref[...]


def kernel(returns, positions, velocities, W1, b1, W2, b2, gamma, beta):
    B = returns.shape[0]
    posP = jnp.full((_PAD, 2), 1e9, jnp.float32).at[:_N].set(positions)
    velP = jnp.zeros((_PAD, 2), jnp.float32).at[:_N].set(velocities)
    psig = jnp.asarray(_PSIG_NP)
    w1t = jnp.zeros((_PAD, 64), jnp.float32).at[0:3].set(W1.T)
    feats, enc = pl.pallas_call(
        functools.partial(_sim_kernel, B),
        out_shape=(jax.ShapeDtypeStruct((B, _PAD), jnp.float32),
                   jax.ShapeDtypeStruct((B, 32), jnp.float32)),
        scratch_shapes=[pltpu.VMEM((B + 8, _PAD), jnp.float32),
                        pltpu.VMEM((B + 8, _PAD), jnp.float32)],
    )(psig, posP, velP, w1t, b1.reshape(1, 64), W2.T, b2.reshape(1, 32),
      gamma.reshape(1, 32), beta.reshape(1, 32))
    return feats[:, 0], feats[:, 1], feats[:, 2], enc
